# baseline (speedup 1.0000x reference)
import jax
import jax.numpy as jnp
from jax.experimental import pallas as pl
from jax.experimental.pallas import tpu as pltpu

G, S, M, E, H, C = 4, 8192, 1024, 8, 4096, 2

BS_A = 1024
NSB_A = S // BS_A
BS_C = 4096
NSB_C = S // BS_C
BH = 1024
NH = H // BH

N_A = G * NSB_A
N_B = E * NH
N_C = G * NSB_C


def _moe_kernel(x_ref, gw_ref, gb_ref, fc1_ref, fc2_ref, out_ref,
                cw_ref, z_ref, eo_ref):
    i = pl.program_id(0)

    @pl.when(i < N_A)
    def _phase_a():
        g = i // NSB_A
        s = i % NSB_A
        x = x_ref[0]
        logits = jax.lax.dot_general(
            gw_ref[...], x, (((1,), (1,)), ((), ())),
            preferred_element_type=jnp.float32) + gb_ref[...]
        m0 = jnp.max(logits, axis=0, keepdims=True)
        ex = jnp.exp(logits - m0)
        scores = ex / jnp.sum(ex, axis=0, keepdims=True)
        eids = jax.lax.broadcasted_iota(jnp.int32, scores.shape, 0)
        v1 = jnp.max(scores, axis=0, keepdims=True)
        first1 = jnp.min(jnp.where(scores == v1, eids, E), axis=0, keepdims=True)
        oh1 = (eids == first1).astype(jnp.float32)
        masked = jnp.where(eids == first1, -jnp.inf, scores)
        v2 = jnp.max(masked, axis=0, keepdims=True)
        first2 = jnp.min(jnp.where(masked == v2, eids, E), axis=0, keepdims=True)
        oh2 = (eids == first2).astype(jnp.float32)
        e2 = jnp.exp(v2 - v1)
        w1 = 1.0 / (1.0 + e2)
        w2 = e2 / (1.0 + e2)
        cw_ref[g, :, pl.ds(s * BS_A, BS_A)] = C * (oh1 * w1 + oh2 * w2)
        dm = oh1 + oh2
        zpart = jax.lax.dot_general(
            dm, x, (((1,), (0,)), ((), ())),
            preferred_element_type=jnp.float32)

        @pl.when(s == 0)
        def _():
            z_ref[g] = zpart

        @pl.when(s != 0)
        def _():
            z_ref[g] += zpart

    @pl.when((i >= N_A) & (i < N_A + N_B))
    def _phase_b():
        j = i - N_A
        e = j // NH
        h_idx = j % NH
        zfull = z_ref[...]
        esel = jax.lax.broadcasted_iota(jnp.int32, zfull.shape, 1) == e
        z = jnp.sum(jnp.where(esel, zfull, 0.0), axis=1)
        f1 = fc1_ref[0]
        f2 = fc2_ref[0]
        h = jax.lax.dot_general(
            z, f1, (((1,), (1,)), ((), ())),
            preferred_element_type=jnp.float32)
        h = jnp.maximum(h, 0.0)
        contrib = jax.lax.dot_general(
            h, f2, (((1,), (1,)), ((), ())),
            preferred_element_type=jnp.float32)

        @pl.when(h_idx == 0)
        def _():
            eo_ref[e] = contrib

        @pl.when(h_idx != 0)
        def _():
            eo_ref[e] += contrib

    @pl.when(i >= N_A + N_B)
    def _phase_c():
        k = i - (N_A + N_B)
        g = k // NSB_C
        s = k % NSB_C
        cw = cw_ref[g, :, pl.ds(s * BS_C, BS_C)]
        eofull = eo_ref[...]
        gsel = jax.lax.broadcasted_iota(jnp.int32, eofull.shape, 1) == g
        eo = jnp.sum(jnp.where(gsel, eofull, 0.0), axis=1)
        out_ref[0] = jax.lax.dot_general(
            cw, eo, (((0,), (0,)), ((), ())),
            preferred_element_type=jnp.float32)


def _x_map(i):
    ia = jnp.minimum(i, N_A - 1)
    return (ia // NSB_A, ia % NSB_A, 0)


def _fc1_map(i):
    j = jnp.clip(i - N_A, 0, N_B - 1)
    return (j // NH, j % NH, 0)


def _fc2_map(i):
    j = jnp.clip(i - N_A, 0, N_B - 1)
    return (j // NH, 0, j % NH)


def _out_map(i):
    k = jnp.clip(i - (N_A + N_B), 0, N_C - 1)
    return (k // NSB_C, k % NSB_C, 0)


@jax.jit
def kernel(x, gate_w, gate_b, fc1, fc2):
    gb = gate_b.reshape(E, 1).astype(jnp.float32)

    out = pl.pallas_call(
        _moe_kernel,
        grid=(N_A + N_B + N_C,),
        in_specs=[
            pl.BlockSpec((1, BS_A, M), _x_map),
            pl.BlockSpec((E, M), lambda i: (0, 0)),
            pl.BlockSpec((E, 1), lambda i: (0, 0)),
            pl.BlockSpec((1, BH, M), _fc1_map),
            pl.BlockSpec((1, M, BH), _fc2_map),
        ],
        out_specs=pl.BlockSpec((1, BS_C, M), _out_map),
        out_shape=jax.ShapeDtypeStruct((G, S, M), jnp.float32),
        compiler_params=pltpu.CompilerParams(
            vmem_limit_bytes=100 * 1024 * 1024),
        scratch_shapes=[
            pltpu.VMEM((G, E, S), jnp.float32),
            pltpu.VMEM((G, E, M), jnp.float32),
            pltpu.VMEM((E, G, M), jnp.float32),
        ],
    )(x, gate_w, gb, fc1, fc2)

    return out


# speedup vs baseline: 1.0655x; 1.0655x over previous
import jax
import jax.numpy as jnp
from jax.experimental import pallas as pl
from jax.experimental.pallas import tpu as pltpu

G, S, M, E, H, C = 4, 8192, 1024, 8, 4096, 2

BS = 2048
NSB = S // BS
BH = 1024
NH = H // BH

N_A = G * NSB
N_B = E * NH
N_C = G * NSB


def _moe_kernel(x_ref, gw_ref, gb_ref, fc1_ref, fc2_ref, out_ref,
                cw_ref, z_ref, eo_ref):
    i = pl.program_id(0)

    @pl.when(i < N_A)
    def _phase_a():
        g = i // NSB
        s = i % NSB
        x = x_ref[0]
        logits = jax.lax.dot_general(
            gw_ref[...], x, (((1,), (1,)), ((), ())),
            preferred_element_type=jnp.float32) + gb_ref[...]
        m0 = jnp.max(logits, axis=0, keepdims=True)
        ex = jnp.exp(logits - m0)
        scores = ex / jnp.sum(ex, axis=0, keepdims=True)
        eids = jax.lax.broadcasted_iota(jnp.int32, scores.shape, 0)
        v1 = jnp.max(scores, axis=0, keepdims=True)
        first1 = jnp.min(jnp.where(scores == v1, eids, E), axis=0, keepdims=True)
        oh1 = (eids == first1).astype(jnp.float32)
        masked = jnp.where(eids == first1, -jnp.inf, scores)
        v2 = jnp.max(masked, axis=0, keepdims=True)
        first2 = jnp.min(jnp.where(masked == v2, eids, E), axis=0, keepdims=True)
        oh2 = (eids == first2).astype(jnp.float32)
        e2 = jnp.exp(v2 - v1)
        w1 = 1.0 / (1.0 + e2)
        w2 = e2 / (1.0 + e2)
        cw_ref[g, :, pl.ds(s * BS, BS)] = C * (oh1 * w1 + oh2 * w2)
        dm = oh1 + oh2
        zpart = jax.lax.dot_general(
            dm, x, (((1,), (0,)), ((), ())),
            preferred_element_type=jnp.float32)

        @pl.when(s == 0)
        def _():
            z_ref[g] = zpart

        @pl.when(s != 0)
        def _():
            z_ref[g] += zpart

    @pl.when((i >= N_A) & (i < N_A + N_B))
    def _phase_b():
        j = i - N_A
        e = j // NH
        h_idx = j % NH
        zfull = z_ref[...]
        esel = jax.lax.broadcasted_iota(jnp.int32, zfull.shape, 1) == e
        z = jnp.sum(jnp.where(esel, zfull, 0.0), axis=1)
        f1 = fc1_ref[0]
        f2 = fc2_ref[0]
        h = jax.lax.dot_general(
            z, f1, (((1,), (1,)), ((), ())),
            preferred_element_type=jnp.float32)
        h = jnp.maximum(h, 0.0)
        contrib = jax.lax.dot_general(
            h, f2, (((1,), (1,)), ((), ())),
            preferred_element_type=jnp.float32)

        @pl.when(h_idx == 0)
        def _():
            eo_ref[e] = contrib

        @pl.when(h_idx != 0)
        def _():
            eo_ref[e] += contrib

    @pl.when(i >= N_A + N_B)
    def _phase_c():
        k = i - (N_A + N_B)
        g = k // NSB
        s = k % NSB
        cw = cw_ref[g, :, pl.ds(s * BS, BS)]
        eofull = eo_ref[...]
        gsel = jax.lax.broadcasted_iota(jnp.int32, eofull.shape, 1) == g
        eo = jnp.sum(jnp.where(gsel, eofull, 0.0), axis=1)
        out_ref[0] = jax.lax.dot_general(
            cw, eo, (((0,), (0,)), ((), ())),
            preferred_element_type=jnp.float32)


def _x_map(i):
    ia = jnp.minimum(i, N_A - 1)
    return (ia // NSB, ia % NSB, 0)


def _fc1_map(i):
    j = jnp.clip(i - N_A, 0, N_B - 1)
    return (j // NH, j % NH, 0)


def _fc2_map(i):
    j = jnp.clip(i - N_A, 0, N_B - 1)
    return (j // NH, 0, j % NH)


def _out_map(i):
    k = jnp.clip(i - (N_A + N_B), 0, N_C - 1)
    return (k // NSB, k % NSB, 0)


@jax.jit
def kernel(x, gate_w, gate_b, fc1, fc2):
    gb = gate_b.reshape(E, 1).astype(jnp.float32)

    out = pl.pallas_call(
        _moe_kernel,
        grid=(N_A + N_B + N_C,),
        in_specs=[
            pl.BlockSpec((1, BS, M), _x_map),
            pl.BlockSpec((E, M), lambda i: (0, 0)),
            pl.BlockSpec((E, 1), lambda i: (0, 0)),
            pl.BlockSpec((1, BH, M), _fc1_map),
            pl.BlockSpec((1, M, BH), _fc2_map),
        ],
        out_specs=pl.BlockSpec((1, BS, M), _out_map),
        out_shape=jax.ShapeDtypeStruct((G, S, M), jnp.float32),
        compiler_params=pltpu.CompilerParams(
            vmem_limit_bytes=100 * 1024 * 1024),
        scratch_shapes=[
            pltpu.VMEM((G, E, S), jnp.float32),
            pltpu.VMEM((G, E, M), jnp.float32),
            pltpu.VMEM((E, G, M), jnp.float32),
        ],
    )(x, gate_w, gb, fc1, fc2)

    return out
